# baseline (speedup 1.0000x reference)
import functools

import jax
import jax.numpy as jnp
from jax import lax
from jax.experimental import pallas as pl
from jax.experimental.pallas import tpu as pltpu
from jax.experimental.pallas import tpu_sc as plsc

N = 10000
E = 320000
D = 128

NC = 2
NS = 16
NW = NC * NS
E_PER_W = E // NW
CHUNK = 80
N_CHUNKS = E_PER_W // CHUNK
RCHUNK = 200
N_RCHUNKS = N // RCHUNK

BLK = 5000
GRID = N // BLK


def _msg_body(x_ref, w_ref, b_ref, o_ref):
    o_ref[...] = jnp.maximum(
        jnp.dot(x_ref[...], w_ref[...], preferred_element_type=jnp.float32)
        + b_ref[...], 0.0)


def _msg_matmul(x, w, b):
    return pl.pallas_call(
        _msg_body,
        grid=(GRID,),
        in_specs=[
            pl.BlockSpec((BLK, D), lambda i: (i, 0)),
            pl.BlockSpec((D, D), lambda i: (0, 0)),
            pl.BlockSpec((1, D), lambda i: (0, 0)),
        ],
        out_specs=pl.BlockSpec((BLK, D), lambda i: (i, 0)),
        out_shape=jax.ShapeDtypeStruct((N, D), jnp.float32),
    )(x, w, b.reshape(1, D))


NBUF = 4
N_BLOCKS = 5
CH_PER_BLK = N_CHUNKS // N_BLOCKS
RINGS_PER_BLK = CH_PER_BLK // NBUF
E_PER_BLK = CH_PER_BLK * CHUNK


def _sc_segment_sum(h, src, dst4, zrows):
    mesh = plsc.VectorSubcoreMesh(
        core_axis_name="c", subcore_axis_name="s",
        num_cores=NC, num_subcores=NS)

    @functools.partial(
        pl.kernel,
        out_type=jax.ShapeDtypeStruct((NC * N, D), jnp.float32),
        mesh=mesh,
        scratch_types=[
            pltpu.VMEM((E_PER_BLK,), jnp.int32),
            pltpu.VMEM((E_PER_BLK,), jnp.int32),
            [pltpu.VMEM((CHUNK, D), jnp.float32) for _ in range(NBUF)],
            pltpu.VMEM_SHARED((N, D), jnp.float32),
            [pltpu.SemaphoreType.DMA for _ in range(NBUF)],
            [pltpu.SemaphoreType.DMA for _ in range(NBUF)],
        ],
    )
    def k(h_hbm, src_hbm, dst_hbm, z_hbm, out_hbm, idx_s, idx_d, rows, acc,
          gsem, ssem):
        ci = lax.axis_index("c")
        si = lax.axis_index("s")
        wid = si * NC + ci

        def zbody(j, carry):
            @pl.when(j % NS == si)
            def _():
                pltpu.sync_copy(z_hbm, acc.at[pl.ds(j * RCHUNK, RCHUNK)])
            return carry

        lax.fori_loop(0, N_RCHUNKS, zbody, 0)
        plsc.subcore_barrier()

        def drain_scatter(b):
            pltpu.make_async_copy(
                rows[b], acc.at[idx_d.at[pl.ds(0, CHUNK)]], ssem[b]).wait()

        def wait_gather(b):
            pltpu.make_async_copy(
                h_hbm.at[idx_s.at[pl.ds(0, CHUNK)]], rows[b], gsem[b]).wait()

        def gather(c, b):
            pltpu.async_copy(
                h_hbm.at[idx_s.at[pl.ds(c * CHUNK, CHUNK)]], rows[b], gsem[b])

        def scatter(c, b):
            pltpu.async_copy(
                rows[b], acc.at[idx_d.at[pl.ds(c * CHUNK, CHUNK)]], ssem[b],
                add=True)

        @pl.loop(0, N_BLOCKS)
        def block(p):
            pltpu.sync_copy(
                src_hbm.at[pl.ds(wid * E_PER_W + p * E_PER_BLK, E_PER_BLK)],
                idx_s)
            pltpu.sync_copy(
                dst_hbm.at[pl.ds(wid * E_PER_W + p * E_PER_BLK, E_PER_BLK)],
                idx_d)

            @pl.loop(0, RINGS_PER_BLK)
            def ring(t):
                base = t * NBUF
                for b in range(NBUF):
                    @pl.when(t > 0)
                    def _(b=b):
                        drain_scatter(b)
                    gather(base + b, b)
                for b in range(NBUF):
                    wait_gather(b)
                    scatter(base + b, b)

            tail = CH_PER_BLK - 1
            drain_scatter(0)
            gather(tail, 0)
            wait_gather(0)
            scatter(tail, 0)
            for b in range(NBUF):
                drain_scatter(b)

        plsc.subcore_barrier()

        def obody(j, carry):
            @pl.when(j % NS == si)
            def _():
                pltpu.sync_copy(
                    acc.at[pl.ds(j * RCHUNK, RCHUNK)],
                    out_hbm.at[pl.ds(ci * N + j * RCHUNK, RCHUNK)])
            return carry

        lax.fori_loop(0, N_RCHUNKS, obody, 0)

    return k(h, src, dst4, zrows)


def _head_body(r_ref, l_ref, p0_ref, p1_ref, wua_ref, wub_ref, we1a_ref,
               we1b_ref, we2_ref, woa_ref, wob_ref, bu_ref, b1_ref, b2_ref,
               bo_ref, o_ref):
    f32 = jnp.float32
    agg = p0_ref[...] + p1_ref[...]
    lt = jnp.maximum(
        jnp.dot(l_ref[...], wua_ref[...], preferred_element_type=f32)
        + jnp.dot(agg, wub_ref[...], preferred_element_type=f32)
        + bu_ref[...], 0.0)
    pre1 = (jnp.dot(r_ref[...], we1a_ref[...], preferred_element_type=f32)
            + jnp.dot(lt, we1b_ref[...], preferred_element_type=f32)
            + b1_ref[...])
    e1 = jnp.where(pre1 >= 0, pre1, 0.01 * pre1)
    e2 = jnp.maximum(
        jnp.dot(lt, we2_ref[...], preferred_element_type=f32) + b2_ref[...],
        0.0)
    fo = jnp.maximum(
        jnp.dot(e1, woa_ref[...], preferred_element_type=f32)
        + jnp.dot(e2, wob_ref[...], preferred_element_type=f32)
        + bo_ref[...], 0.0)
    prefactor = 1.380649e-23 * 353.0 / 6.62607015e-34 / 3.6
    o_ref[...] = prefactor * jnp.exp(-fo * 4184.0 / 8.31 / 353.0)


def _head(reactant, ligand, partials, W_upd, b_upd, W_e1, b_e1, W_e2, b_e2,
          W_out, b_out):
    full = lambda shape: pl.BlockSpec(shape, lambda i: tuple(0 for _ in shape))
    return pl.pallas_call(
        _head_body,
        grid=(GRID,),
        in_specs=[
            pl.BlockSpec((BLK, D), lambda i: (i, 0)),
            pl.BlockSpec((BLK, D), lambda i: (i, 0)),
            pl.BlockSpec((BLK, D), lambda i: (i, 0)),
            pl.BlockSpec((BLK, D), lambda i: (i + GRID, 0)),
            full((D, D)),
            full((D, D)),
            full((D, 256)),
            full((D, 256)),
            full((D, D)),
            full((256, 384)),
            full((D, 384)),
            full((1, D)),
            full((1, 256)),
            full((1, D)),
            full((1, 384)),
        ],
        out_specs=pl.BlockSpec((BLK, 384), lambda i: (i, 0)),
        out_shape=jax.ShapeDtypeStruct((N, 384), jnp.float32),
    )(reactant, ligand, partials, partials,
      W_upd[:D], W_upd[D:], W_e1[:D], W_e1[D:], W_e2,
      W_out[:256], W_out[256:],
      b_upd.reshape(1, D), b_e1.reshape(1, 256), b_e2.reshape(1, D),
      b_out.reshape(1, 384))


def kernel(reactant_data, ligand_data, edge_index, W_msg, b_msg, W_upd, b_upd,
           W_e1, b_e1, W_e2, b_e2, W_out, b_out):
    h = _msg_matmul(ligand_data, W_msg, b_msg)
    zrows = jnp.zeros((RCHUNK, D), jnp.float32)
    partials = _sc_segment_sum(h, edge_index[0], edge_index[1], zrows)
    return _head(reactant_data, ligand_data, partials, W_upd, b_upd,
                 W_e1, b_e1, W_e2, b_e2, W_out, b_out)


# speedup vs baseline: 1.0266x; 1.0266x over previous
import functools

import jax
import jax.numpy as jnp
from jax import lax
from jax.experimental import pallas as pl
from jax.experimental.pallas import tpu as pltpu
from jax.experimental.pallas import tpu_sc as plsc

N = 10000
E = 320000
D = 128

NC = 2
NS = 16
NW = NC * NS
E_PER_W = E // NW
CHUNK = 80
N_CHUNKS = E_PER_W // CHUNK
RCHUNK = 200
N_RCHUNKS = N // RCHUNK

BLK = 2000
GRID = N // BLK


def _msg_body(x_ref, w_ref, b_ref, o_ref):
    o_ref[...] = jnp.maximum(
        jnp.dot(x_ref[...], w_ref[...], preferred_element_type=jnp.float32)
        + b_ref[...], 0.0)


def _msg_matmul(x, w, b):
    return pl.pallas_call(
        _msg_body,
        grid=(GRID,),
        in_specs=[
            pl.BlockSpec((BLK, D), lambda i: (i, 0)),
            pl.BlockSpec((D, D), lambda i: (0, 0)),
            pl.BlockSpec((1, D), lambda i: (0, 0)),
        ],
        out_specs=pl.BlockSpec((BLK, D), lambda i: (i, 0)),
        out_shape=jax.ShapeDtypeStruct((N, D), jnp.float32),
    )(x, w, b.reshape(1, D))


NBUF = 4
N_BLOCKS = 5
CH_PER_BLK = N_CHUNKS // N_BLOCKS
RINGS_PER_BLK = CH_PER_BLK // NBUF
E_PER_BLK = CH_PER_BLK * CHUNK


def _sc_segment_sum(h, src, dst, zrows):
    mesh = plsc.VectorSubcoreMesh(
        core_axis_name="c", subcore_axis_name="s",
        num_cores=NC, num_subcores=NS)

    @functools.partial(
        pl.kernel,
        out_type=jax.ShapeDtypeStruct((NC * N, D), jnp.float32),
        mesh=mesh,
        scratch_types=[
            pltpu.VMEM((E_PER_BLK,), jnp.int32),
            pltpu.VMEM((E_PER_BLK,), jnp.int32),
            [pltpu.VMEM((CHUNK, D), jnp.float32) for _ in range(NBUF)],
            pltpu.VMEM_SHARED((N, D), jnp.float32),
            [pltpu.SemaphoreType.DMA for _ in range(NBUF)],
            [pltpu.SemaphoreType.DMA for _ in range(NBUF)],
        ],
    )
    def k(h_hbm, src_hbm, dst_hbm, z_hbm, out_hbm, idx_s, idx_d, rows, acc,
          gsem, ssem):
        ci = lax.axis_index("c")
        si = lax.axis_index("s")
        wid = si * NC + ci

        def zbody(j, carry):
            @pl.when(j % NS == si)
            def _():
                pltpu.sync_copy(z_hbm, acc.at[pl.ds(j * RCHUNK, RCHUNK)])
            return carry

        lax.fori_loop(0, N_RCHUNKS, zbody, 0)
        plsc.subcore_barrier()

        def drain_scatter(b):
            pltpu.make_async_copy(
                rows[b], acc.at[idx_d.at[pl.ds(0, CHUNK)]], ssem[b]).wait()

        def wait_gather(b):
            pltpu.make_async_copy(
                h_hbm.at[idx_s.at[pl.ds(0, CHUNK)]], rows[b], gsem[b]).wait()

        def gather(c, b):
            pltpu.async_copy(
                h_hbm.at[idx_s.at[pl.ds(c * CHUNK, CHUNK)]], rows[b], gsem[b])

        def scatter(c, b):
            pltpu.async_copy(
                rows[b], acc.at[idx_d.at[pl.ds(c * CHUNK, CHUNK)]], ssem[b],
                add=True)

        @pl.loop(0, N_BLOCKS)
        def block(p):
            ds_i = pltpu.async_copy(
                src_hbm.at[pl.ds(wid * E_PER_W + p * E_PER_BLK, E_PER_BLK)],
                idx_s, gsem[0])
            dd_i = pltpu.async_copy(
                dst_hbm.at[pl.ds(wid * E_PER_W + p * E_PER_BLK, E_PER_BLK)],
                idx_d, gsem[1])
            ds_i.wait()
            dd_i.wait()

            @pl.loop(0, RINGS_PER_BLK)
            def ring(t):
                base = t * NBUF
                for b in range(NBUF):
                    @pl.when(t > 0)
                    def _(b=b):
                        drain_scatter(b)
                    gather(base + b, b)
                for b in range(NBUF):
                    wait_gather(b)
                    scatter(base + b, b)

            tail = CH_PER_BLK - 1
            drain_scatter(0)
            gather(tail, 0)
            wait_gather(0)
            scatter(tail, 0)
            for b in range(NBUF):
                drain_scatter(b)

        plsc.subcore_barrier()

        def obody(j, carry):
            @pl.when(j % NS == si)
            def _():
                pltpu.sync_copy(
                    acc.at[pl.ds(j * RCHUNK, RCHUNK)],
                    out_hbm.at[pl.ds(ci * N + j * RCHUNK, RCHUNK)])
            return carry

        lax.fori_loop(0, N_RCHUNKS, obody, 0)

    return k(h, src, dst, zrows)


def _head_body(r_ref, l_ref, p0_ref, p1_ref, wua_ref, wub_ref, we1a_ref,
               we1b_ref, we2_ref, woa_ref, wob_ref, bu_ref, b1_ref, b2_ref,
               bo_ref, o_ref):
    f32 = jnp.float32
    agg = p0_ref[...] + p1_ref[...]
    lt = jnp.maximum(
        jnp.dot(l_ref[...], wua_ref[...], preferred_element_type=f32)
        + jnp.dot(agg, wub_ref[...], preferred_element_type=f32)
        + bu_ref[...], 0.0)
    pre1 = (jnp.dot(r_ref[...], we1a_ref[...], preferred_element_type=f32)
            + jnp.dot(lt, we1b_ref[...], preferred_element_type=f32)
            + b1_ref[...])
    e1 = jnp.where(pre1 >= 0, pre1, 0.01 * pre1)
    e2 = jnp.maximum(
        jnp.dot(lt, we2_ref[...], preferred_element_type=f32) + b2_ref[...],
        0.0)
    fo = jnp.maximum(
        jnp.dot(e1, woa_ref[...], preferred_element_type=f32)
        + jnp.dot(e2, wob_ref[...], preferred_element_type=f32)
        + bo_ref[...], 0.0)
    prefactor = 1.380649e-23 * 353.0 / 6.62607015e-34 / 3.6
    o_ref[...] = prefactor * jnp.exp(-fo * 4184.0 / 8.31 / 353.0)


def _head(reactant, ligand, partials, W_upd, b_upd, W_e1, b_e1, W_e2, b_e2,
          W_out, b_out):
    full = lambda shape: pl.BlockSpec(shape, lambda i: tuple(0 for _ in shape))
    return pl.pallas_call(
        _head_body,
        grid=(GRID,),
        in_specs=[
            pl.BlockSpec((BLK, D), lambda i: (i, 0)),
            pl.BlockSpec((BLK, D), lambda i: (i, 0)),
            pl.BlockSpec((BLK, D), lambda i: (i, 0)),
            pl.BlockSpec((BLK, D), lambda i: (i + GRID, 0)),
            full((D, D)),
            full((D, D)),
            full((D, 256)),
            full((D, 256)),
            full((D, D)),
            full((256, 384)),
            full((D, 384)),
            full((1, D)),
            full((1, 256)),
            full((1, D)),
            full((1, 384)),
        ],
        out_specs=pl.BlockSpec((BLK, 384), lambda i: (i, 0)),
        out_shape=jax.ShapeDtypeStruct((N, 384), jnp.float32),
    )(reactant, ligand, partials, partials,
      W_upd[:D], W_upd[D:], W_e1[:D], W_e1[D:], W_e2,
      W_out[:256], W_out[256:],
      b_upd.reshape(1, D), b_e1.reshape(1, 256), b_e2.reshape(1, D),
      b_out.reshape(1, 384))


def kernel(reactant_data, ligand_data, edge_index, W_msg, b_msg, W_upd, b_upd,
           W_e1, b_e1, W_e2, b_e2, W_out, b_out):
    h = _msg_matmul(ligand_data, W_msg, b_msg)
    zrows = jnp.zeros((RCHUNK, D), jnp.float32)
    partials = _sc_segment_sum(h, edge_index[0], edge_index[1], zrows)
    return _head(reactant_data, ligand_data, partials, W_upd, b_upd,
                 W_e1, b_e1, W_e2, b_e2, W_out, b_out)
